# baseline (speedup 1.0000x reference)
import jax
import jax.numpy as jnp
from jax.experimental import pallas as pl

N = 10000
E = 320000
NC = 1000
EC = 32000
M = 320000
D = 128
L = 2


def _pallas_mm(a, w, block_rows=400):
    nb = a.shape[0] // block_rows

    def body(a_ref, w_ref, o_ref):
        o_ref[...] = jnp.dot(a_ref[...], w_ref[...],
                             preferred_element_type=jnp.float32)

    return pl.pallas_call(
        body,
        grid=(nb,),
        in_specs=[pl.BlockSpec((block_rows, a.shape[1]), lambda i: (i, 0)),
                  pl.BlockSpec(w.shape, lambda i: (0, 0))],
        out_specs=pl.BlockSpec((block_rows, w.shape[1]), lambda i: (i, 0)),
        out_shape=jax.ShapeDtypeStruct((a.shape[0], w.shape[1]), jnp.float32),
    )(a, w)


def _mlp_ref(h, ws):
    n = len(ws) // 2
    for i in range(n):
        h = h @ ws[2 * i] + ws[2 * i + 1]
        if i < n - 1:
            h = jax.nn.relu(h)
    return h


def _inorm_ref(h):
    mu = h.mean(0, keepdims=True)
    var = h.var(0, keepdims=True)
    return (h - mu) / jnp.sqrt(var + 1e-5)


def _seg(vals, idx, n):
    return jax.ops.segment_sum(vals, idx, num_segments=n)


def _tag_fine(c, src, dst, w, Ws, b):
    out = c @ Ws[0]
    h = _seg(w[:, None] * c[src], dst, N)
    out = out + _pallas_mm(h, Ws[1])
    h = _seg(w[:, None] * h[src], dst, N)
    out = out + _pallas_mm(h, Ws[2])
    return out + b


def _tag_coarse(c, src, dst, w, Ws, b):
    out = c @ Ws[0]
    h = _seg(w[:, None] * c[src], dst, NC)
    out = out + _pallas_mm(h, Ws[1], block_rows=NC)
    h = _seg(w[:, None] * h[src], dst, NC)
    out = out + _pallas_mm(h, Ws[2], block_rows=NC)
    return out + b


def kernel(x, edge_index, edge_attr, coarse_edge_index, attr_coarse,
           f2c_assign, r_vals, mask_row, mask_col, params):
    src0, dst0 = edge_index[0], edge_index[1]
    src1, dst1 = coarse_edge_index[0], coarse_edge_index[1]

    ea_main = _mlp_ref(edge_attr, params['pem'])
    x1 = _seg(r_vals[:, None] * x, f2c_assign, NC)
    h0 = _mlp_ref(x, params['pn'])
    h1 = _mlp_ref(x1, params['pn'])
    w0 = _mlp_ref(ea_main, params['pe'])[:, 0]
    w1 = _mlp_ref(attr_coarse, params['pe'])[:, 0]

    deg1 = _seg(jnp.ones((N,), jnp.float32), f2c_assign, NC)
    mask1 = (deg1 > 0).astype(x.dtype)[:, None]
    for l in range(L):
        c0 = jnp.concatenate([h0, jax.nn.relu(h0)], axis=1)
        c1 = jnp.concatenate([h1, jax.nn.relu(h1) * mask1], axis=1)
        h0 = _tag_fine(c0, src0, dst0, w0,
                       params['tag_W'][l, 0], params['tag_b'][l, 0])
        h1 = _tag_coarse(c1, src1, dst1, w1,
                         params['tag_W'][l, 1], params['tag_b'][l, 1])
        h0 = _inorm_ref(jax.nn.relu(h0))
        h1 = _inorm_ref(jax.nn.relu(h1))
    xc = h1 @ params['loc_W'] + params['loc_b']
    xf = h0 @ params['lo_W'] + params['lo_b']

    eR_in = jnp.concatenate([xc[f2c_assign], xf], axis=1)
    eR = _mlp_ref(eR_in, params['emr'])
    out_R = jnp.zeros((NC, N), jnp.float32).at[
        f2c_assign, jnp.arange(N)].add(eR[:, 0])

    eM_in = jnp.concatenate([xf[mask_row], xf[mask_col]], axis=1)
    eM = _mlp_ref(eM_in, params['em'])
    out = jnp.zeros((N, N), jnp.float32).at[mask_row, mask_col].add(eM[:, 0])
    return (out, out_R)
